# baseline (speedup 1.0000x reference)
import jax
import jax.numpy as jnp
from jax.experimental import pallas as pl

_B, _F, _C = 1024, 26, 1000
_BLK = 16


def _onehot_body(x_ref, o_ref):
    iota = jax.lax.broadcasted_iota(jnp.int32, (_BLK, _C), 1)
    for f in range(_F):
        o_ref[:, f * _C:(f + 1) * _C] = (x_ref[:, f:f + 1] == iota).astype(
            jnp.int32)


def kernel(x):
    return pl.pallas_call(
        _onehot_body,
        grid=(_B // _BLK,),
        in_specs=[pl.BlockSpec((_BLK, _F), lambda i: (i, 0))],
        out_specs=pl.BlockSpec((_BLK, _F * _C), lambda i: (i, 0)),
        out_shape=jax.ShapeDtypeStruct((_B, _F * _C), jnp.int32),
    )(x)


# speedup vs baseline: 1.5473x; 1.5473x over previous
import numpy as np
import jax
import jax.numpy as jnp
from jax.experimental import pallas as pl

_B, _F, _C = 1024, 26, 1000
_N = _F * _C
_BLK = 32

_S = jnp.asarray(
    (np.arange(_F)[:, None] == (np.arange(_N) // _C)).astype(np.float32))
_JMOD = jnp.asarray((np.arange(_N) % _C).astype(np.float32)[None, :])


def _onehot_body(x_ref, s_ref, jmod_ref, o_ref):
    xf = x_ref[...].astype(jnp.float32)
    k = jnp.dot(xf, s_ref[...], preferred_element_type=jnp.float32)
    o_ref[...] = (k == jmod_ref[...]).astype(jnp.int32)


def kernel(x):
    return pl.pallas_call(
        _onehot_body,
        grid=(_B // _BLK,),
        in_specs=[
            pl.BlockSpec((_BLK, _F), lambda i: (i, 0)),
            pl.BlockSpec((_F, _N), lambda i: (0, 0)),
            pl.BlockSpec((1, _N), lambda i: (0, 0)),
        ],
        out_specs=pl.BlockSpec((_BLK, _N), lambda i: (i, 0)),
        out_shape=jax.ShapeDtypeStruct((_B, _N), jnp.int32),
    )(x, _S, _JMOD)
